# Initial kernel scaffold; baseline (speedup 1.0000x reference)
import jax
import jax.numpy as jnp
from jax.experimental import pallas as pl


def kernel(signal, window):
    raise NotImplementedError("write your pallas kernel here")


# speedup vs baseline: 209.8762x; 209.8762x over previous
import functools
import math

import jax
import jax.numpy as jnp
from jax.experimental import pallas as pl
from jax.experimental.pallas import tpu as pltpu

_N_FFT = 512
_HOP = 128
_F = _N_FFT // 2 + 1


def _body(x_ref, w_ref, sst_ref, mag_ref, dd_ref, *, n_frames):
    b = pl.program_id(0)
    a = x_ref[0]
    frames = jnp.concatenate(
        [a[:, 0:n_frames], a[:, 1:n_frames + 1],
         a[:, 2:n_frames + 2], a[:, 3:n_frames + 3]], axis=0)
    spec = jax.lax.dot_general(
        w_ref[...], frames, (((1,), (0,)), ((), ())),
        precision=jax.lax.Precision.HIGHEST,
        preferred_element_type=jnp.float32)
    re = spec[0:_F, :]
    im = spec[_F:2 * _F, :]
    mag = jnp.sqrt(re * re + im * im)
    mag_ref[0] = mag

    @pl.when(b == 0)
    def _compute_displacement():
        pr = re[:, 1:] * re[:, :-1] + im[:, 1:] * im[:, :-1]
        pi = im[:, 1:] * re[:, :-1] - re[:, 1:] * im[:, :-1]
        pr = jnp.concatenate([pr[:, :1], pr], axis=1)
        pi = jnp.concatenate([pi[:, :1], pi], axis=1)
        apr = jnp.abs(pr)
        api = jnp.abs(pi)
        d = jnp.where(
            api <= pr, 0,
            jnp.where(pi > apr, 1,
                      jnp.where(-pi > apr, -1,
                                jnp.where(pi >= 0, 2, -2)))).astype(jnp.int32)
        fio = jax.lax.broadcasted_iota(jnp.int32, (_F, n_frames), 0)
        dd_ref[...] = jnp.clip(fio + d, 0, _F - 1) - fio

    dd = dd_ref[...]
    fio = jax.lax.broadcasted_iota(jnp.int32, (_F, n_frames), 0)
    valid = jnp.logical_and(fio >= 1, fio <= _F - 2)
    acc = jnp.zeros((_F, n_frames), jnp.float32)
    for k in (-2, -1, 0, 1, 2):
        c = jnp.where(jnp.logical_and(valid, dd == k), mag, 0.0)
        if k > 0:
            c = jnp.concatenate(
                [jnp.zeros((k, n_frames), jnp.float32), c[:_F - k]], axis=0)
        elif k < 0:
            c = jnp.concatenate(
                [c[-k:], jnp.zeros((-k, n_frames), jnp.float32)], axis=0)
        acc = acc + c
    sst_ref[0] = acc


def kernel(signal, window):
    B, L = signal.shape
    pad = _N_FFT // 2
    x = jnp.pad(signal, ((0, 0), (pad, pad)), mode="reflect")
    n_frames = 1 + (x.shape[1] - _N_FFT) // _HOP
    n_chunks = x.shape[1] // _HOP
    xT = jnp.transpose(x.reshape(B, n_chunks, _HOP), (0, 2, 1))

    kn = (jnp.arange(_F, dtype=jnp.int32)[:, None]
          * jnp.arange(_N_FFT, dtype=jnp.int32)[None, :]) % _N_FFT
    ang = (2.0 * math.pi / _N_FFT) * kn.astype(jnp.float32)
    wr = jnp.cos(ang) * window[None, :]
    wi = -jnp.sin(ang) * window[None, :]
    W = jnp.concatenate([wr, wi], axis=0)

    out_shape = [
        jax.ShapeDtypeStruct((B, _F, n_frames), jnp.float32),
        jax.ShapeDtypeStruct((B, _F, n_frames), jnp.float32),
    ]
    sst, mag = pl.pallas_call(
        functools.partial(_body, n_frames=n_frames),
        grid=(B,),
        in_specs=[
            pl.BlockSpec((1, _HOP, n_chunks), lambda b: (b, 0, 0)),
            pl.BlockSpec((2 * _F, _N_FFT), lambda b: (0, 0)),
        ],
        out_specs=[
            pl.BlockSpec((1, _F, n_frames), lambda b: (b, 0, 0)),
            pl.BlockSpec((1, _F, n_frames), lambda b: (b, 0, 0)),
        ],
        out_shape=out_shape,
        scratch_shapes=[pltpu.VMEM((_F, n_frames), jnp.int32)],
        compiler_params=pltpu.CompilerParams(
            dimension_semantics=("arbitrary",)),
    )(xT, W)
    return (sst, mag)


# speedup vs baseline: 257.8968x; 1.2288x over previous
import functools

import jax
import jax.numpy as jnp
import numpy as np
from jax.experimental import pallas as pl
from jax.experimental.pallas import tpu as pltpu

_N_FFT = 512
_HOP = 128
_F = _N_FFT // 2 + 1

_KN = (np.arange(_F, dtype=np.int64)[:, None]
       * np.arange(_N_FFT, dtype=np.int64)[None, :]) % _N_FFT
_ANG = (2.0 * np.pi / _N_FFT) * _KN.astype(np.float64)
_DFT = np.concatenate(
    [np.cos(_ANG), -np.sin(_ANG)], axis=0).astype(np.float32)


def _body(x_ref, w_ref, sst_ref, mag_ref, dd_ref, *, n_frames):
    b = pl.program_id(0)
    a = x_ref[0]
    frames = jnp.concatenate(
        [a[:, 0:n_frames], a[:, 1:n_frames + 1],
         a[:, 2:n_frames + 2], a[:, 3:n_frames + 3]], axis=0)
    spec = jax.lax.cond(
        b == 0,
        lambda: jax.lax.dot_general(
            w_ref[...], frames, (((1,), (0,)), ((), ())),
            precision=jax.lax.Precision.HIGHEST,
            preferred_element_type=jnp.float32),
        lambda: jax.lax.dot_general(
            w_ref[...], frames, (((1,), (0,)), ((), ())),
            precision=jax.lax.Precision.DEFAULT,
            preferred_element_type=jnp.float32))
    re = spec[0:_F, :]
    im = spec[_F:2 * _F, :]
    mag = jnp.sqrt(re * re + im * im)
    mag_ref[0] = mag

    @pl.when(b == 0)
    def _compute_displacement():
        pr = re[:, 1:] * re[:, :-1] + im[:, 1:] * im[:, :-1]
        pi = im[:, 1:] * re[:, :-1] - re[:, 1:] * im[:, :-1]
        pr = jnp.concatenate([pr[:, :1], pr], axis=1)
        pi = jnp.concatenate([pi[:, :1], pi], axis=1)
        apr = jnp.abs(pr)
        api = jnp.abs(pi)
        d = jnp.where(
            api <= pr, 0,
            jnp.where(pi > apr, 1,
                      jnp.where(-pi > apr, -1,
                                jnp.where(pi >= 0, 2, -2)))).astype(jnp.int32)
        fio = jax.lax.broadcasted_iota(jnp.int32, (_F, n_frames), 0)
        dd_ref[...] = jnp.clip(fio + d, 0, _F - 1) - fio

    dd = dd_ref[...]
    fio = jax.lax.broadcasted_iota(jnp.int32, (_F, n_frames), 0)
    valid = jnp.logical_and(fio >= 1, fio <= _F - 2)
    acc = jnp.zeros((_F, n_frames), jnp.float32)
    for k in (-2, -1, 0, 1, 2):
        c = jnp.where(jnp.logical_and(valid, dd == k), mag, 0.0)
        if k > 0:
            c = jnp.concatenate(
                [jnp.zeros((k, n_frames), jnp.float32), c[:_F - k]], axis=0)
        elif k < 0:
            c = jnp.concatenate(
                [c[-k:], jnp.zeros((-k, n_frames), jnp.float32)], axis=0)
        acc = acc + c
    sst_ref[0] = acc


def kernel(signal, window):
    B, L = signal.shape
    pad = _N_FFT // 2
    x = jnp.pad(signal, ((0, 0), (pad, pad)), mode="reflect")
    n_frames = 1 + (x.shape[1] - _N_FFT) // _HOP
    n_chunks = x.shape[1] // _HOP
    xT = jnp.transpose(x.reshape(B, n_chunks, _HOP), (0, 2, 1))

    W = jnp.asarray(_DFT) * window[None, :]

    out_shape = [
        jax.ShapeDtypeStruct((B, _F, n_frames), jnp.float32),
        jax.ShapeDtypeStruct((B, _F, n_frames), jnp.float32),
    ]
    sst, mag = pl.pallas_call(
        functools.partial(_body, n_frames=n_frames),
        grid=(B,),
        in_specs=[
            pl.BlockSpec((1, _HOP, n_chunks), lambda b: (b, 0, 0)),
            pl.BlockSpec((2 * _F, _N_FFT), lambda b: (0, 0)),
        ],
        out_specs=[
            pl.BlockSpec((1, _F, n_frames), lambda b: (b, 0, 0)),
            pl.BlockSpec((1, _F, n_frames), lambda b: (b, 0, 0)),
        ],
        out_shape=out_shape,
        scratch_shapes=[pltpu.VMEM((_F, n_frames), jnp.int32)],
        compiler_params=pltpu.CompilerParams(
            dimension_semantics=("arbitrary",)),
    )(xT, W)
    return (sst, mag)


# speedup vs baseline: 301.2237x; 1.1680x over previous
import functools

import jax
import jax.numpy as jnp
import numpy as np
from jax.experimental import pallas as pl
from jax.experimental.pallas import tpu as pltpu

_N_FFT = 512
_HOP = 128
_F = _N_FFT // 2 + 1
_NT = 4

_KN = (np.arange(_F, dtype=np.int64)[:, None]
       * np.arange(_N_FFT, dtype=np.int64)[None, :]) % _N_FFT
_ANG = (2.0 * np.pi / _N_FFT) * _KN.astype(np.float64)
_DFT = np.concatenate(
    [np.cos(_ANG), -np.sin(_ANG)], axis=0).astype(np.float32)

_SENTINEL = 99


def _body(x_ref, e_ref, w_ref, sst_ref, mag_ref, spec_ref, dd_ref,
          *, n_frames, t_block):
    b = pl.program_id(0)
    t = pl.program_id(1)

    @pl.when(t == 0)
    def _compute_spectrum():
        araw = jnp.transpose(x_ref[0], (1, 0))
        et = e_ref[0]
        a = jnp.concatenate([et[:, 0:2], araw, et[:, 2:4]], axis=1)
        frames = jnp.concatenate(
            [a[:, 0:n_frames], a[:, 1:n_frames + 1],
             a[:, 2:n_frames + 2], a[:, 3:n_frames + 3]], axis=0)
        dims = (((1,), (0,)), ((), ()))
        spec_ref[:, 0:n_frames] = jax.lax.dot_general(
            w_ref[...], frames, dims,
            precision=jax.lax.Precision.DEFAULT,
            preferred_element_type=jnp.float32)

        @pl.when(b == 0)
        def _compute_displacement():
            wf = w_ref[...]
            wh = wf.astype(jnp.bfloat16)
            wl = (wf - wh.astype(jnp.float32)).astype(jnp.bfloat16)
            fh = frames.astype(jnp.bfloat16)
            fl = (frames - fh.astype(jnp.float32)).astype(jnp.bfloat16)
            spec_hi = (
                jax.lax.dot_general(wh, fh, dims,
                                    preferred_element_type=jnp.float32)
                + jax.lax.dot_general(wh, fl, dims,
                                      preferred_element_type=jnp.float32)
                + jax.lax.dot_general(wl, fh, dims,
                                      preferred_element_type=jnp.float32))
            reh = spec_hi[0:_F, :]
            imh = spec_hi[_F:2 * _F, :]
            pr = reh[:, 1:] * reh[:, :-1] + imh[:, 1:] * imh[:, :-1]
            pi = imh[:, 1:] * reh[:, :-1] - reh[:, 1:] * imh[:, :-1]
            pr = jnp.concatenate([pr[:, :1], pr], axis=1)
            pi = jnp.concatenate([pi[:, :1], pi], axis=1)
            apr = jnp.abs(pr)
            api = jnp.abs(pi)
            d = jnp.where(
                api <= pr, 0,
                jnp.where(pi > apr, 1,
                          jnp.where(-pi > apr, -1,
                                    jnp.where(pi >= 0, 2,
                                              -2)))).astype(jnp.int32)
            fio = jax.lax.broadcasted_iota(jnp.int32, (_F, n_frames), 0)
            valid = jnp.logical_and(fio >= 1, fio <= _F - 2)
            dd_ref[:, 0:n_frames] = jnp.where(
                valid, jnp.clip(fio + d, 0, _F - 1) - fio,
                _SENTINEL).astype(jnp.bfloat16)

    t0 = t * t_block
    spec = spec_ref[:, pl.ds(t0, t_block)]
    re = spec[0:_F, :]
    im = spec[_F:2 * _F, :]
    mag = jnp.sqrt(re * re + im * im)
    mag_ref[0] = mag

    dd = dd_ref[:, pl.ds(t0, t_block)]
    magb = mag.astype(jnp.bfloat16)
    acc = jnp.zeros((_F, t_block), jnp.bfloat16)
    for k in (-2, -1, 0, 1, 2):
        c = jnp.where(dd == k, magb, jnp.bfloat16(0))
        if k > 0:
            c = jnp.concatenate(
                [jnp.zeros((k, t_block), jnp.bfloat16), c[:_F - k]], axis=0)
        elif k < 0:
            c = jnp.concatenate(
                [c[-k:], jnp.zeros((-k, t_block), jnp.bfloat16)], axis=0)
        acc = acc + c
    sst_ref[0] = acc.astype(jnp.float32)


def kernel(signal, window):
    B, L = signal.shape
    pad = _N_FFT // 2
    n_frames = 1 + L // _HOP
    t_block = -(-n_frames // (_NT * 128)) * 128
    n_slices = -(-n_frames // t_block)
    t_pad = n_slices * t_block
    xc = signal.reshape(B, L // _HOP, _HOP)
    edges = jnp.concatenate(
        [signal[:, pad:0:-1], signal[:, L - 2:L - 2 - pad:-1]],
        axis=1).reshape(B, 4, _HOP)
    edges = jnp.pad(jnp.transpose(edges, (0, 2, 1)), ((0, 0), (0, 0), (0, 4)))

    W = jnp.asarray(_DFT) * window[None, :]

    out_shape = [
        jax.ShapeDtypeStruct((B, _F, n_frames), jnp.float32),
        jax.ShapeDtypeStruct((B, _F, n_frames), jnp.float32),
    ]
    sst, mag = pl.pallas_call(
        functools.partial(_body, n_frames=n_frames, t_block=t_block),
        grid=(B, n_slices),
        in_specs=[
            pl.BlockSpec((1, L // _HOP, _HOP), lambda b, t: (b, 0, 0)),
            pl.BlockSpec((1, _HOP, 8), lambda b, t: (b, 0, 0)),
            pl.BlockSpec((2 * _F, _N_FFT), lambda b, t: (0, 0)),
        ],
        out_specs=[
            pl.BlockSpec((1, _F, t_block), lambda b, t: (b, 0, t)),
            pl.BlockSpec((1, _F, t_block), lambda b, t: (b, 0, t)),
        ],
        out_shape=out_shape,
        scratch_shapes=[
            pltpu.VMEM((2 * _F, t_pad), jnp.float32),
            pltpu.VMEM((_F, t_pad), jnp.bfloat16),
        ],
        compiler_params=pltpu.CompilerParams(
            dimension_semantics=("arbitrary", "arbitrary"),
            vmem_limit_bytes=120 * 1024 * 1024),
    )(xc, edges, W)
    return (sst, mag)
